# baseline (speedup 1.0000x reference)
import functools

import jax
import jax.numpy as jnp
from jax import lax
from jax.experimental import pallas as pl
from jax.experimental.pallas import tpu as pltpu
from jax.experimental.pallas import tpu_sc as plsc

_G = 16
_CHUNK = 1024


def _make_kernel(M, L, C, B):
    info = plsc.get_sparse_core_info()
    NC, NS = info.num_cores, info.num_subcores
    NW = NC * NS
    assert M % NW == 0
    tpw = M // NW
    gpc = _CHUNK // _G
    nch = B // _CHUNK

    mesh = plsc.VectorSubcoreMesh(core_axis_name="c", subcore_axis_name="s")

    @functools.partial(
        pl.kernel,
        mesh=mesh,
        compiler_params=pltpu.CompilerParams(
            use_tc_tiling_on_sc=True, needs_layout_passes=False),
        out_type=jax.ShapeDtypeStruct((M, C, B), jnp.float32),
        scratch_types=[
            pltpu.VMEM((B,), jnp.int32),
            pltpu.VMEM((3, _G, C, 128), jnp.float32),
            pltpu.VMEM((C, _CHUNK), jnp.float32),
            pltpu.SemaphoreType.DMA((3,)),
        ],
    )
    def k(table_hbm, idx_hbm, out_hbm, idx_v, blk_v, out_v, sem):
        wid = lax.axis_index("s") * NC + lax.axis_index("c")
        ci = lax.iota(jnp.int32, C)

        def issue(t, g, buf):
            lv = idx_v[pl.ds(g * _G, _G)]
            for u in range(_G):
                li = lv[u]
                col0 = pl.multiple_of((li >> 7) << 7, 128)
                pltpu.async_copy(
                    table_hbm.at[t, :, pl.ds(col0, 128)],
                    blk_v.at[buf, u], sem.at[buf])
            return lv & 127

        def drain(t, buf):
            pltpu.make_async_copy(
                table_hbm.at[pl.ds(0, _G), :, pl.ds(0, 128)],
                blk_v.at[buf], sem.at[buf]).wait()

        def process(buf, offv, lb):
            for u in range(_G):
                vals = plsc.load_gather(
                    blk_v,
                    [lax.broadcast(buf, (C,)),
                     jnp.full((C,), u, jnp.int32), ci,
                     lax.broadcast(offv[u], (C,))],
                )
                plsc.store_scatter(
                    out_v, [ci, lb + u + jnp.zeros((C,), jnp.int32)], vals)

        def tree_body(tl, carry):
            t = wid * tpw + tl
            pltpu.sync_copy(idx_hbm.at[t], idx_v)

            def chunk_body(ch, carry):
                g0 = ch * gpc
                glast = g0 + gpc - 1
                off0 = issue(t, g0, jnp.int32(0))
                off1 = issue(t, g0 + 1, jnp.int32(1))

                def grp_body(p, offs):
                    off_a, off_b = offs
                    r = lax.rem(p, 3)
                    r2 = lax.rem(p + 2, 3)
                    drain(t, r)
                    offn = issue(t, jnp.minimum(g0 + p + 2, glast), r2)
                    process(r, off_a, p * _G)
                    return (off_b, offn)

                lax.fori_loop(0, gpc, grp_body, (off0, off1))
                drain(t, lax.rem(jnp.int32(gpc), 3))
                drain(t, lax.rem(jnp.int32(gpc + 1), 3))
                pltpu.sync_copy(
                    out_v, out_hbm.at[t, :, pl.ds(ch * _CHUNK, _CHUNK)])
                return carry

            lax.fori_loop(0, nch, chunk_body, 0)
            return carry

        lax.fori_loop(0, tpw, tree_body, 0)

    return k


def kernel(x, idx, leafs):
    M, L, C = leafs.shape
    B = idx.shape[0]
    table = leafs.transpose(0, 2, 1)
    out_t = _make_kernel(M, L, C, B)(table, idx.T)
    return out_t.transpose(0, 2, 1)


# speedup vs baseline: 1.0043x; 1.0043x over previous
import functools

import jax
import jax.numpy as jnp
from jax import lax
from jax.experimental import pallas as pl
from jax.experimental.pallas import tpu as pltpu
from jax.experimental.pallas import tpu_sc as plsc

_G = 16
_CHUNK = 1024


def _make_kernel(M, L, C, B):
    info = plsc.get_sparse_core_info()
    NC, NS = info.num_cores, info.num_subcores
    NW = NC * NS
    assert M % NW == 0
    tpw = M // NW
    gpc = _CHUNK // _G
    nch = B // _CHUNK

    mesh = plsc.VectorSubcoreMesh(core_axis_name="c", subcore_axis_name="s")

    @functools.partial(
        pl.kernel,
        mesh=mesh,
        compiler_params=pltpu.CompilerParams(
            use_tc_tiling_on_sc=True, needs_layout_passes=False),
        out_type=jax.ShapeDtypeStruct((M, C, B), jnp.float32),
        scratch_types=[
            pltpu.VMEM((B,), jnp.int32),
            pltpu.VMEM((3, _G, C, 128), jnp.float32),
            pltpu.VMEM((C, _CHUNK), jnp.float32),
            pltpu.SemaphoreType.DMA((3,)),
        ],
    )
    def k(table_hbm, idx_hbm, out_hbm, idx_v, blk_v, out_v, sem):
        wid = lax.axis_index("s") * NC + lax.axis_index("c")
        ci = lax.iota(jnp.int32, C)

        def issue(t, g, buf):
            lv = idx_v[pl.ds(g * _G, _G)]
            for u in range(_G):
                li = lv[u]
                col0 = pl.multiple_of((li >> 7) << 7, 128)
                pltpu.async_copy(
                    table_hbm.at[t, :, pl.ds(col0, 128)],
                    blk_v.at[buf, u], sem.at[buf])
            return lv & 127

        def drain(t, buf):
            for u in range(_G):
                pltpu.make_async_copy(
                    table_hbm.at[t, :, pl.ds(0, 128)],
                    blk_v.at[buf, u], sem.at[buf]).wait()

        def process(buf, offv, lb):
            for u in range(_G):
                vals = plsc.load_gather(
                    blk_v,
                    [lax.broadcast(buf, (C,)),
                     jnp.full((C,), u, jnp.int32), ci,
                     lax.broadcast(offv[u], (C,))],
                )
                plsc.store_scatter(
                    out_v, [ci, lb + u + jnp.zeros((C,), jnp.int32)], vals)

        def tree_body(tl, carry):
            t = wid * tpw + tl
            pltpu.sync_copy(idx_hbm.at[t], idx_v)

            def chunk_body(ch, carry):
                g0 = ch * gpc
                glast = g0 + gpc - 1
                off0 = issue(t, g0, jnp.int32(0))
                off1 = issue(t, g0 + 1, jnp.int32(1))

                def grp_body(p, offs):
                    off_a, off_b = offs
                    r = lax.rem(p, 3)
                    r2 = lax.rem(p + 2, 3)
                    drain(t, r)
                    offn = issue(t, jnp.minimum(g0 + p + 2, glast), r2)
                    process(r, off_a, p * _G)
                    return (off_b, offn)

                lax.fori_loop(0, gpc, grp_body, (off0, off1))
                drain(t, lax.rem(jnp.int32(gpc), 3))
                drain(t, lax.rem(jnp.int32(gpc + 1), 3))
                pltpu.sync_copy(
                    out_v, out_hbm.at[t, :, pl.ds(ch * _CHUNK, _CHUNK)])
                return carry

            lax.fori_loop(0, nch, chunk_body, 0)
            return carry

        lax.fori_loop(0, tpw, tree_body, 0)

    return k


def kernel(x, idx, leafs):
    M, L, C = leafs.shape
    B = idx.shape[0]
    table = leafs.transpose(0, 2, 1)
    out_t = _make_kernel(M, L, C, B)(table, idx.T)
    return out_t.transpose(0, 2, 1)
